# baseline (speedup 1.0000x reference)
import functools

import jax
import jax.numpy as jnp
from jax import lax
from jax.experimental import pallas as pl
from jax.experimental.pallas import tpu as pltpu
from jax.experimental.pallas import tpu_sc as plsc

NUM_EMBEDDINGS = 1000000
EMBED_DIM = 64
BATCH = 16384
NUM_CORES = 2
NUM_SUBCORES = 16
NUM_WORKERS = NUM_CORES * NUM_SUBCORES
B_PER_W = BATCH // NUM_WORKERS
LANES = 16
PAIR = 2 * EMBED_DIM
N_PAIR_ROWS = NUM_EMBEDDINGS // 2
CHUNK = 128
N_CHUNKS = B_PER_W // CHUNK

_mesh = plsc.VectorSubcoreMesh(core_axis_name="c", subcore_axis_name="s")


@functools.partial(
    pl.kernel,
    mesh=_mesh,
    out_type=jax.ShapeDtypeStruct((BATCH, EMBED_DIM), jnp.float32),
    scratch_types=[
        pltpu.VMEM((B_PER_W,), jnp.int32),
        pltpu.VMEM((B_PER_W,), jnp.int32),
        pltpu.VMEM((2, CHUNK, PAIR), jnp.float32),
        pltpu.VMEM((B_PER_W, EMBED_DIM), jnp.float32),
        pltpu.SemaphoreType.DMA((2,)),
    ],
)
def _gather_sc(idx_hbm, wpairs_hbm, out_hbm, idx_v, pidx_v, gbuf, out_stage,
               gsem):
    wid = lax.axis_index("s") * NUM_CORES + lax.axis_index("c")
    base = wid * B_PER_W
    pltpu.sync_copy(idx_hbm.at[pl.ds(base, B_PER_W)], idx_v)

    @pl.loop(0, B_PER_W // LANES)
    def _pair_idx(g):
        v = idx_v[pl.ds(g * LANES, LANES)]
        pidx_v[pl.ds(g * LANES, LANES)] = jax.lax.shift_right_logical(v, 1)

    def start_gather(c):
        return pltpu.async_copy(
            wpairs_hbm.at[pidx_v.at[pl.ds(c * CHUNK, CHUNK)]],
            gbuf.at[c % 2],
            gsem.at[c % 2],
        )

    gather = start_gather(0)
    for c in range(N_CHUNKS):
        gather.wait()
        next_gather = start_gather(c + 1) if c + 1 < N_CHUNKS else None
        buf = c % 2

        @pl.loop(0, CHUNK // LANES)
        def _group(g):
            k0 = c * CHUNK + g * LANES
            ids = idx_v[pl.ds(k0, LANES)]
            half16 = jax.lax.bitwise_and(ids, 1) * EMBED_DIM
            for j in range(LANES):
                off = half16[j]
                kloc = g * LANES + j
                kglob = k0 + j
                for q in range(EMBED_DIM // LANES):
                    vals = gbuf[buf, kloc, pl.ds(off + q * LANES, LANES)]
                    out_stage[kglob, pl.ds(q * LANES, LANES)] = vals

        gather = next_gather

    pltpu.sync_copy(out_stage, out_hbm.at[pl.ds(base, B_PER_W)])


def kernel(batch, w):
    idx = batch.astype(jnp.int32)
    w_pairs = jnp.reshape(w, (N_PAIR_ROWS, PAIR))
    return _gather_sc(idx, w_pairs)


# speedup vs baseline: 1.5274x; 1.5274x over previous
import functools

import jax
import jax.numpy as jnp
from jax import lax
from jax.experimental import pallas as pl
from jax.experimental.pallas import tpu as pltpu

NUM_EMBEDDINGS = 1000000
EMBED_DIM = 64
BATCH = 16384
NSEM = 8


def _make_tc(n_rows, nsem=8, unroll=8):
    groups = n_rows // nsem

    def body(idx_s, w_hbm, out_hbm, buf, sems):
        def issue(o, _):
            for j in range(nsem):
                i = o * nsem + j
                r = idx_s[i]
                pltpu.make_async_copy(
                    w_hbm.at[pl.ds(r, 1)],
                    buf.at[pl.ds(i, 1)],
                    sems.at[j],
                ).start()
            return 0

        lax.fori_loop(0, groups, issue, 0, unroll=unroll)
        for j in range(nsem):
            pltpu.make_async_copy(
                w_hbm.at[pl.ds(0, groups)],
                buf.at[pl.ds(j * groups, groups)],
                sems.at[j],
            ).wait()
        pltpu.sync_copy(buf, out_hbm)

    return pl.pallas_call(
        body,
        out_shape=jax.ShapeDtypeStruct((n_rows, EMBED_DIM), jnp.float32),
        in_specs=[
            pl.BlockSpec(memory_space=pltpu.SMEM),
            pl.BlockSpec(memory_space=pl.ANY),
        ],
        out_specs=pl.BlockSpec(memory_space=pl.ANY),
        scratch_shapes=[
            pltpu.VMEM((n_rows, EMBED_DIM), jnp.float32),
            pltpu.SemaphoreType.DMA((nsem,)),
        ],
    )


_gather_tc = _make_tc(BATCH)


def kernel(batch, w):
    return _gather_tc(batch.astype(jnp.int32), w)
